# baseline (speedup 1.0000x reference)
import functools

import jax
import jax.numpy as jnp
from jax import lax
from jax.experimental import pallas as pl
from jax.experimental.pallas import tpu as pltpu
from jax.experimental.pallas import tpu_sc as plsc

NC = 2
NS = 16
NW = NC * NS
EDGE_BATCH = 128


def _make_sc_aggregate(n_pad, e, h, with_count):
  assert e % (NW * EDGE_BATCH) == 0
  nb = e // (NW * EDGE_BATCH)
  assert n_pad % (NS * 16) == 0
  rows_per_tile = n_pad // NS

  mesh = plsc.VectorSubcoreMesh(
      core_axis_name="c", subcore_axis_name="s",
      num_cores=NC, num_subcores=NS)

  NBUF = 6
  PREF = 3
  assert nb >= NBUF

  out_type = [jax.ShapeDtypeStruct((NC * n_pad, h), jnp.float32)]
  scratch = [
      pltpu.VMEM_SHARED((n_pad, h), jnp.float32),
      pltpu.VMEM((nb, EDGE_BATCH), jnp.int32),
      pltpu.VMEM((nb, EDGE_BATCH), jnp.int32),
      [pltpu.VMEM((EDGE_BATCH, h), jnp.float32) for _ in range(NBUF)],
      pltpu.VMEM((rows_per_tile, h), jnp.float32),
      [pltpu.SemaphoreType.DMA for _ in range(NBUF)],
      [pltpu.SemaphoreType.DMA for _ in range(NBUF)],
  ]
  if with_count:
    out_type.append(jax.ShapeDtypeStruct((NC * n_pad,), jnp.float32))
    scratch += [
        pltpu.VMEM_SHARED((n_pad,), jnp.float32),
        pltpu.VMEM((EDGE_BATCH,), jnp.float32),
        pltpu.VMEM((rows_per_tile,), jnp.float32),
        pltpu.SemaphoreType.DMA,
    ]

  def body(feat_hbm, src_hbm, dst_hbm, agg_hbm, *rest):
    if with_count:
      (cnt_hbm, agg_sh, src_l, dst_l, rows, stage_v, gsem, ssem,
       cnt_sh, ones_v, cstage_v, csem) = rest
    else:
      agg_sh, src_l, dst_l, rows, stage_v, gsem, ssem = rest
    cid = lax.axis_index("c")
    sid = lax.axis_index("s")
    wid = sid * NC + cid
    row0 = sid * rows_per_tile

    pltpu.sync_copy(src_hbm.at[pl.ds(wid * nb, nb)], src_l)
    pltpu.sync_copy(dst_hbm.at[pl.ds(wid * nb, nb)], dst_l)

    def zrow(i, _):
      stage_v[i] = jnp.zeros((h,), jnp.float32)
      return 0
    lax.fori_loop(0, rows_per_tile, zrow, 0)
    pltpu.sync_copy(stage_v, agg_sh.at[pl.ds(row0, rows_per_tile)])
    if with_count:
      def zc(i, _):
        cstage_v[pl.ds(i * 16, 16)] = jnp.zeros((16,), jnp.float32)
        return 0
      lax.fori_loop(0, rows_per_tile // 16, zc, 0)
      pltpu.sync_copy(cstage_v, cnt_sh.at[pl.ds(row0, rows_per_tile)])
      def oc(i, _):
        ones_v[pl.ds(i * 16, 16)] = jnp.ones((16,), jnp.float32)
        return 0
      lax.fori_loop(0, EDGE_BATCH // 16, oc, 0)
    plsc.subcore_barrier()

    def start_gather(b, j):
      pltpu.async_copy(feat_hbm.at[src_l.at[b]], rows[j], gsem[j])

    def wait_gather(j):
      pltpu.make_async_copy(feat_hbm.at[src_l.at[0]], rows[j], gsem[j]).wait()

    def start_scatter(b, j):
      pltpu.async_copy(rows[j], agg_sh.at[dst_l.at[b]], ssem[j], add=True)
      if with_count:
        pltpu.async_copy(ones_v, cnt_sh.at[dst_l.at[b]], csem, add=True)

    def wait_scatter(j):
      pltpu.make_async_copy(rows[j], agg_sh.at[dst_l.at[0]], ssem[j]).wait()

    def pipe_body(b, j, static_tail):
      jn = (j + PREF) % NBUF
      wait_gather(j)
      start_scatter(b, j)
      bn = b + PREF
      if static_tail:
        if bn < nb:
          if bn >= NBUF:
            wait_scatter(jn)
          start_gather(bn, jn)
      else:
        @pl.when(bn < nb)
        def _():
          wait_scatter(jn)
          start_gather(bn, jn)

    for b in range(PREF):
      start_gather(b, b)
    n_groups = nb // NBUF
    for j in range(NBUF):
      pipe_body(j, j, static_tail=True)

    def group(g, _):
      b0 = g * NBUF
      for j in range(NBUF):
        pipe_body(b0 + j, j, static_tail=False)
      return 0
    lax.fori_loop(1, n_groups, group, 0)
    for b in range(n_groups * NBUF, nb):
      pipe_body(b, b % NBUF, static_tail=True)

    for b in range(nb - NBUF, nb):
      wait_scatter(b % NBUF)
    if with_count:
      def cdrain(i, _):
        pltpu.make_async_copy(ones_v, cnt_sh.at[dst_l.at[0]], csem).wait()
        return 0
      lax.fori_loop(0, nb, cdrain, 0)
    plsc.subcore_barrier()

    out0 = cid * n_pad + row0
    pltpu.sync_copy(agg_sh.at[pl.ds(row0, rows_per_tile)], stage_v)
    pltpu.sync_copy(stage_v, agg_hbm.at[pl.ds(out0, rows_per_tile)])
    if with_count:
      pltpu.sync_copy(cnt_sh.at[pl.ds(row0, rows_per_tile)], cstage_v)
      pltpu.sync_copy(cstage_v, cnt_hbm.at[pl.ds(out0, rows_per_tile)])

  return pl.kernel(
      body, out_type=out_type, mesh=mesh, scratch_types=scratch,
      compiler_params=pltpu.CompilerParams(
          use_tc_tiling_on_sc=False,
          disable_bounds_checks=True,
          disable_semaphore_checks=True))


def _tc_project(x, w_l, w_r, b):
  n, _ = x.shape
  hh = w_l.shape[1]

  def body(x_ref, wl_ref, wr_ref, b_ref, xl_ref, xr_ref):
    xv = x_ref[...]
    xl_ref[...] = jnp.dot(xv, wl_ref[...], preferred_element_type=jnp.float32)
    xr_ref[...] = (jnp.dot(xv, wr_ref[...], preferred_element_type=jnp.float32)
                   + b_ref[...])

  return pl.pallas_call(
      body,
      out_shape=[jax.ShapeDtypeStruct((n, hh), jnp.float32),
                 jax.ShapeDtypeStruct((n, hh), jnp.float32)],
  )(x, w_l, w_r, b)


def _tc_mid(n, agg, cnt, xr, w2l_p, w2r_p, b2_p):
  hh = agg.shape[-1]

  def body(agg_ref, cnt_ref, xr_ref, wl_ref, wr_ref, b_ref, hl_ref, hr_ref):
    a = agg_ref[0, :n, :] + agg_ref[1, :n, :]
    c = cnt_ref[0, :n, :] + cnt_ref[1, :n, :]
    mean = a / jnp.clip(c, 1.0)
    hv = jnp.maximum(mean + xr_ref[...], 0.0)
    hl_ref[...] = jnp.dot(hv, wl_ref[...], preferred_element_type=jnp.float32)
    hr_ref[...] = (jnp.dot(hv, wr_ref[...], preferred_element_type=jnp.float32)
                   + b_ref[...])

  return pl.pallas_call(
      body,
      out_shape=[jax.ShapeDtypeStruct((n, hh), jnp.float32),
                 jax.ShapeDtypeStruct((n, hh), jnp.float32)],
  )(agg, cnt, xr, w2l_p, w2r_p, b2_p)


def _tc_post(n, c_dim, agg, cnt, hr):
  hh = agg.shape[-1]

  def body(agg_ref, cnt_ref, hr_ref, out_ref):
    a = agg_ref[0, :n, :] + agg_ref[1, :n, :]
    c = cnt_ref[0, :n, :] + cnt_ref[1, :n, :]
    z = a / jnp.clip(c, 1.0) + hr_ref[...]
    col = lax.broadcasted_iota(jnp.int32, (n, hh), 1)
    valid = col < c_dim
    zm = jnp.where(valid, z, -jnp.inf)
    m = jnp.max(zm, axis=1, keepdims=True)
    ez = jnp.where(valid, jnp.exp(z - m), 0.0)
    lse = jnp.log(jnp.sum(ez, axis=1, keepdims=True)) + m
    out_ref[...] = z - lse

  return pl.pallas_call(
      body,
      out_shape=jax.ShapeDtypeStruct((n, hh), jnp.float32),
  )(agg, cnt, hr)


def kernel(x, edge_index, W1_l, W1_r, b1, W2_l, W2_r, b2):
  n, d = x.shape
  e = edge_index.shape[1]
  hh = W1_l.shape[1]
  c_dim = W2_l.shape[1]
  n_pad = ((n + NS * 16) // (NS * 16)) * (NS * 16)

  assert e % EDGE_BATCH == 0
  rows = e // EDGE_BATCH
  rows_pad = ((rows + NW - 1) // NW) * NW
  n_extra = rows_pad - rows
  src = edge_index[0].astype(jnp.int32).reshape(rows, EDGE_BATCH)
  dst = edge_index[1].astype(jnp.int32).reshape(rows, EDGE_BATCH)
  if n_extra:
    fill = jnp.arange(n_extra * EDGE_BATCH, dtype=jnp.int32)
    src_fill = (fill % n).reshape(n_extra, EDGE_BATCH)
    dst_fill = (n + fill % (n_pad - n)).reshape(n_extra, EDGE_BATCH)
    src = jnp.concatenate([src, src_fill], axis=0)
    dst = jnp.concatenate([dst, dst_fill], axis=0)
  e_pad = rows_pad * EDGE_BATCH

  w2l_p = jnp.zeros((hh, hh), jnp.float32).at[:, :c_dim].set(W2_l)
  w2r_p = jnp.zeros((hh, hh), jnp.float32).at[:, :c_dim].set(W2_r)
  b2_p = jnp.zeros((1, hh), jnp.float32).at[0, :c_dim].set(b2)

  agg1_fn = _make_sc_aggregate(n_pad, e_pad, hh, with_count=True)
  agg2_fn = _make_sc_aggregate(n_pad, e_pad, hh, with_count=False)

  xl, xr = _tc_project(x, W1_l, W1_r, b1.reshape(1, hh))
  agg1, cnt = agg1_fn(xl, src, dst)
  agg1 = agg1.reshape(NC, n_pad, hh)
  cnt = cnt.reshape(NC, n_pad, 1)
  hl, hr = _tc_mid(n, agg1, cnt, xr, w2l_p, w2r_p, b2_p)
  (agg2,) = agg2_fn(hl, src, dst)
  agg2 = agg2.reshape(NC, n_pad, hh)
  out = _tc_post(n, c_dim, agg2, cnt, hr)
  return out[:, :c_dim]


# speedup vs baseline: 1.1676x; 1.1676x over previous
import functools

import jax
import jax.numpy as jnp
from jax import lax
from jax.experimental import pallas as pl
from jax.experimental.pallas import tpu as pltpu
from jax.experimental.pallas import tpu_sc as plsc

NC = 2
NS = 16
NW = NC * NS
EDGE_BATCH = 128


def _make_sc_aggregate(n_pad, e, h, with_count):
  assert e % (NW * EDGE_BATCH) == 0
  nb = e // (NW * EDGE_BATCH)
  assert n_pad % (NS * 16) == 0
  rows_per_tile = n_pad // NS

  mesh = plsc.VectorSubcoreMesh(
      core_axis_name="c", subcore_axis_name="s",
      num_cores=NC, num_subcores=NS)

  NBUF = 6
  PREF = 3
  assert nb >= NBUF

  out_type = [jax.ShapeDtypeStruct((NC * n_pad, h), jnp.float32)]
  scratch = [
      pltpu.VMEM_SHARED((n_pad, h), jnp.float32),
      pltpu.VMEM((nb, EDGE_BATCH), jnp.int32),
      pltpu.VMEM((nb, EDGE_BATCH), jnp.int32),
      [pltpu.VMEM((EDGE_BATCH, h), jnp.float32) for _ in range(NBUF)],
      pltpu.VMEM((rows_per_tile, h), jnp.float32),
      [pltpu.SemaphoreType.DMA for _ in range(NBUF)],
      [pltpu.SemaphoreType.DMA for _ in range(NBUF)],
  ]
  if with_count:
    out_type.append(jax.ShapeDtypeStruct((NC * n_pad,), jnp.float32))
    scratch += [
        pltpu.VMEM_SHARED((n_pad,), jnp.float32),
        pltpu.VMEM((EDGE_BATCH,), jnp.float32),
        pltpu.VMEM((rows_per_tile,), jnp.float32),
        pltpu.SemaphoreType.DMA,
    ]

  def body(feat_hbm, src_hbm, dst_hbm, agg_hbm, *rest):
    if with_count:
      (cnt_hbm, agg_sh, src_l, dst_l, rows, stage_v, gsem, ssem,
       cnt_sh, ones_v, cstage_v, csem) = rest
    else:
      agg_sh, src_l, dst_l, rows, stage_v, gsem, ssem = rest
    cid = lax.axis_index("c")
    sid = lax.axis_index("s")
    wid = sid * NC + cid
    row0 = sid * rows_per_tile

    pltpu.sync_copy(src_hbm.at[pl.ds(wid * nb, nb)], src_l)
    pltpu.sync_copy(dst_hbm.at[pl.ds(wid * nb, nb)], dst_l)

    def zrow(i, _):
      stage_v[i] = jnp.zeros((h,), jnp.float32)
      return 0
    lax.fori_loop(0, rows_per_tile, zrow, 0)
    pltpu.sync_copy(stage_v, agg_sh.at[pl.ds(row0, rows_per_tile)])
    if with_count:
      def zc(i, _):
        cstage_v[pl.ds(i * 16, 16)] = jnp.zeros((16,), jnp.float32)
        return 0
      lax.fori_loop(0, rows_per_tile // 16, zc, 0)
      pltpu.sync_copy(cstage_v, cnt_sh.at[pl.ds(row0, rows_per_tile)])
      def oc(i, _):
        ones_v[pl.ds(i * 16, 16)] = jnp.ones((16,), jnp.float32)
        return 0
      lax.fori_loop(0, EDGE_BATCH // 16, oc, 0)
    plsc.subcore_barrier()

    def start_gather(b, j):
      pltpu.async_copy(feat_hbm.at[src_l.at[b]], rows[j], gsem[j])

    def wait_gather(j):
      pltpu.make_async_copy(feat_hbm.at[src_l.at[0]], rows[j], gsem[j]).wait()

    def start_scatter(b, j):
      pltpu.async_copy(rows[j], agg_sh.at[dst_l.at[b]], ssem[j], add=True)
      if with_count:
        pltpu.async_copy(ones_v, cnt_sh.at[dst_l.at[b]], csem, add=True)

    def wait_scatter(j):
      pltpu.make_async_copy(rows[j], agg_sh.at[dst_l.at[0]], ssem[j]).wait()

    def pipe_body(b, j, static_tail):
      jn = (j + PREF) % NBUF
      wait_gather(j)
      start_scatter(b, j)
      bn = b + PREF
      if static_tail:
        if bn < nb:
          if bn >= NBUF:
            wait_scatter(jn)
          start_gather(bn, jn)
      else:
        @pl.when(bn < nb)
        def _():
          wait_scatter(jn)
          start_gather(bn, jn)

    for b in range(PREF):
      start_gather(b, b)
    n_groups = nb // NBUF
    for j in range(NBUF):
      pipe_body(j, j, static_tail=True)

    def group(g, _):
      b0 = g * NBUF
      for j in range(NBUF):
        pipe_body(b0 + j, j, static_tail=False)
      return 0
    lax.fori_loop(1, n_groups, group, 0)
    for b in range(n_groups * NBUF, nb):
      pipe_body(b, b % NBUF, static_tail=True)

    for b in range(nb - NBUF, nb):
      wait_scatter(b % NBUF)
    if with_count:
      def cdrain(i, _):
        pltpu.make_async_copy(ones_v, cnt_sh.at[dst_l.at[0]], csem).wait()
        return 0
      lax.fori_loop(0, nb, cdrain, 0)
    plsc.subcore_barrier()

    out0 = cid * n_pad + row0
    pltpu.sync_copy(agg_sh.at[pl.ds(row0, rows_per_tile)], stage_v)
    pltpu.sync_copy(stage_v, agg_hbm.at[pl.ds(out0, rows_per_tile)])
    if with_count:
      pltpu.sync_copy(cnt_sh.at[pl.ds(row0, rows_per_tile)], cstage_v)
      pltpu.sync_copy(cstage_v, cnt_hbm.at[pl.ds(out0, rows_per_tile)])

  return pl.kernel(
      body, out_type=out_type, mesh=mesh, scratch_types=scratch,
      compiler_params=pltpu.CompilerParams(
          use_tc_tiling_on_sc=False,
          disable_bounds_checks=True,
          disable_semaphore_checks=True))


def _tc_project(np8, xp, wbig_l, wbig_r, b1_tile):
  def body(x_ref, wl_ref, wr_ref, b_ref, xl_ref, xr_ref):
    xv = x_ref[...]
    xl_ref[...] = jnp.dot(xv, wl_ref[...], preferred_element_type=jnp.float32)
    xr_ref[...] = (jnp.dot(xv, wr_ref[...], preferred_element_type=jnp.float32)
                   + b_ref[...])

  return pl.pallas_call(
      body,
      out_shape=[jax.ShapeDtypeStruct((np8, 128), jnp.float32),
                 jax.ShapeDtypeStruct((np8, 128), jnp.float32)],
  )(xp, wbig_l, wbig_r, b1_tile)


def _tc_mid(agg, cnt, xr_p, expand, w2l_bd, w2r_bd, b2_tile):
  np8 = agg.shape[1]

  def body(agg_ref, cnt_ref, xr_ref, e_ref, wl_ref, wr_ref, b_ref,
           hl_ref, hr_ref):
    a = agg_ref[0] + agg_ref[1]
    c = cnt_ref[0] + cnt_ref[1]
    inv = 1.0 / jnp.clip(c, 1.0)
    mean = a * jnp.dot(inv, e_ref[...], preferred_element_type=jnp.float32)
    hv = jnp.maximum(mean + xr_ref[...], 0.0)
    hl_ref[...] = jnp.dot(hv, wl_ref[...], preferred_element_type=jnp.float32)
    hr_ref[...] = (jnp.dot(hv, wr_ref[...], preferred_element_type=jnp.float32)
                   + b_ref[...])

  return pl.pallas_call(
      body,
      out_shape=[jax.ShapeDtypeStruct((np8, 128), jnp.float32),
                 jax.ShapeDtypeStruct((np8, 128), jnp.float32)],
  )(agg, cnt, xr_p, expand, w2l_bd, w2r_bd, b2_tile)


def _tc_post(agg, cnt, hr_p, expand, gsum):
  np8 = agg.shape[1]

  def body(agg_ref, cnt_ref, hr_ref, e_ref, g_ref, out_ref):
    a = agg_ref[0] + agg_ref[1]
    c = cnt_ref[0] + cnt_ref[1]
    inv = 1.0 / jnp.clip(c, 1.0)
    z = (a * jnp.dot(inv, e_ref[...], preferred_element_type=jnp.float32)
         + hr_ref[...])
    m = jnp.max(z, axis=1, keepdims=True)
    ez = jnp.exp(z - m)
    s = jnp.dot(ez, g_ref[...], preferred_element_type=jnp.float32)
    out_ref[...] = (z - m) - jnp.dot(jnp.log(s), e_ref[...],
                                     preferred_element_type=jnp.float32)

  return pl.pallas_call(
      body,
      out_shape=jax.ShapeDtypeStruct((np8, 128), jnp.float32),
  )(agg, cnt, hr_p, expand, gsum)


def kernel(x, edge_index, W1_l, W1_r, b1, W2_l, W2_r, b2):
  n, d = x.shape
  e = edge_index.shape[1]
  hh = W1_l.shape[1]
  c_dim = W2_l.shape[1]
  n_pad = ((n + NS * 16) // (NS * 16)) * (NS * 16)

  assert e % EDGE_BATCH == 0
  rows = e // EDGE_BATCH
  rows_pad = ((rows + NW - 1) // NW) * NW
  n_extra = rows_pad - rows
  src = edge_index[0].astype(jnp.int32).reshape(rows, EDGE_BATCH)
  dst = edge_index[1].astype(jnp.int32).reshape(rows, EDGE_BATCH)
  if n_extra:
    fill = jnp.arange(n_extra * EDGE_BATCH, dtype=jnp.int32)
    src_fill = (fill % n).reshape(n_extra, EDGE_BATCH)
    dst_fill = (n + fill % (n_pad - n)).reshape(n_extra, EDGE_BATCH)
    src = jnp.concatenate([src, src_fill], axis=0)
    dst = jnp.concatenate([dst, dst_fill], axis=0)
  e_pad = rows_pad * EDGE_BATCH

  np8 = n_pad // 8
  gi = jnp.arange(8)
  wbig_l = jnp.zeros((8 * d, 128), jnp.float32)
  wbig_r = jnp.zeros((8 * d, 128), jnp.float32)
  w2l_bd = jnp.zeros((128, 128), jnp.float32)
  w2r_bd = jnp.zeros((128, 128), jnp.float32)
  for g in range(8):
    wbig_l = wbig_l.at[d * g:d * (g + 1), hh * g:hh * g + hh].set(W1_l)
    wbig_r = wbig_r.at[d * g:d * (g + 1), hh * g:hh * g + hh].set(W1_r)
    w2l_bd = w2l_bd.at[hh * g:hh * g + hh, hh * g:hh * g + c_dim].set(W2_l)
    w2r_bd = w2r_bd.at[hh * g:hh * g + hh, hh * g:hh * g + c_dim].set(W2_r)
  b1_tile = jnp.tile(b1, 8).reshape(1, 128)
  b2_tile = jnp.tile(jnp.zeros((hh,), jnp.float32).at[:c_dim].set(b2),
                     8).reshape(1, 128)
  lane = jnp.arange(128)
  expand = (lane[None, :] // hh == gi[:, None]).astype(jnp.float32)
  gsum = ((lane[:, None] // hh == gi[None, :])
          & (lane[:, None] % hh < c_dim)).astype(jnp.float32)

  assert n % 8 == 0 and d == 128
  xp = jnp.pad(x.reshape(n // 8, 8 * d), ((0, np8 - n // 8), (0, 0)))

  agg1_fn = _make_sc_aggregate(n_pad, e_pad, hh, with_count=True)
  agg2_fn = _make_sc_aggregate(n_pad, e_pad, hh, with_count=False)

  xl_p, xr_p = _tc_project(np8, xp, wbig_l, wbig_r, b1_tile)
  agg1, cnt = agg1_fn(xl_p.reshape(n_pad, hh), src, dst)
  agg1 = agg1.reshape(NC, np8, 128)
  cnt = cnt.reshape(NC, np8, 8)
  hl_p, hr_p = _tc_mid(agg1, cnt, xr_p, expand, w2l_bd, w2r_bd, b2_tile)
  (agg2,) = agg2_fn(hl_p.reshape(n_pad, hh), src, dst)
  agg2 = agg2.reshape(NC, np8, 128)
  out_p = _tc_post(agg2, cnt, hr_p, expand, gsum)
  return out_p.reshape(n_pad, hh)[:n, :c_dim]


# speedup vs baseline: 1.2779x; 1.0944x over previous
import functools

import jax
import jax.numpy as jnp
from jax import lax
from jax.experimental import pallas as pl
from jax.experimental.pallas import tpu as pltpu
from jax.experimental.pallas import tpu_sc as plsc

NC = 2
NS = 16
NW = NC * NS
EDGE_BATCH = 128


def _make_sc_aggregate(n_pad, e, h, with_count):
  assert e % (NW * EDGE_BATCH) == 0
  nb = e // (NW * EDGE_BATCH)
  assert n_pad % (NS * 16) == 0
  rows_per_tile = n_pad // NS

  mesh = plsc.VectorSubcoreMesh(
      core_axis_name="c", subcore_axis_name="s",
      num_cores=NC, num_subcores=NS)

  NBUF = 6
  PREF = 3
  assert nb >= NBUF

  out_type = [jax.ShapeDtypeStruct((NC * n_pad, h), jnp.float32)]
  scratch = [
      pltpu.VMEM_SHARED((n_pad, h), jnp.float32),
      pltpu.VMEM((nb, EDGE_BATCH), jnp.int32),
      pltpu.VMEM((nb, EDGE_BATCH), jnp.int32),
      [pltpu.VMEM((EDGE_BATCH, h), jnp.float32) for _ in range(NBUF)],
      pltpu.VMEM((rows_per_tile, h), jnp.float32),
      [pltpu.SemaphoreType.DMA for _ in range(NBUF)],
      [pltpu.SemaphoreType.DMA for _ in range(NBUF)],
  ]
  if with_count:
    out_type.append(jax.ShapeDtypeStruct((NC * n_pad,), jnp.float32))
    scratch += [
        pltpu.VMEM_SHARED((n_pad,), jnp.float32),
        pltpu.VMEM((EDGE_BATCH,), jnp.float32),
        pltpu.VMEM((rows_per_tile,), jnp.float32),
        pltpu.SemaphoreType.DMA,
    ]

  def body(feat_hbm, src_hbm, dst_hbm, agg_hbm, *rest):
    if with_count:
      (cnt_hbm, agg_sh, src_l, dst_l, rows, stage_v, gsem, ssem,
       cnt_sh, ones_v, cstage_v, csem) = rest
    else:
      agg_sh, src_l, dst_l, rows, stage_v, gsem, ssem = rest
    cid = lax.axis_index("c")
    sid = lax.axis_index("s")
    wid = sid * NC + cid
    row0 = sid * rows_per_tile

    pltpu.sync_copy(src_hbm.at[pl.ds(wid * nb, nb)], src_l)
    pltpu.sync_copy(dst_hbm.at[pl.ds(wid * nb, nb)], dst_l)

    def zrow(i, _):
      stage_v[i] = jnp.zeros((h,), jnp.float32)
      return 0
    lax.fori_loop(0, rows_per_tile, zrow, 0)
    pltpu.sync_copy(stage_v, agg_sh.at[pl.ds(row0, rows_per_tile)])
    if with_count:
      def zc(i, _):
        cstage_v[pl.ds(i * 16, 16)] = jnp.zeros((16,), jnp.float32)
        return 0
      lax.fori_loop(0, rows_per_tile // 16, zc, 0)
      pltpu.sync_copy(cstage_v, cnt_sh.at[pl.ds(row0, rows_per_tile)])
      def oc(i, _):
        ones_v[pl.ds(i * 16, 16)] = jnp.ones((16,), jnp.float32)
        return 0
      lax.fori_loop(0, EDGE_BATCH // 16, oc, 0)
    plsc.subcore_barrier()

    def start_gather(b, j):
      pltpu.async_copy(feat_hbm.at[src_l.at[b]], rows[j], gsem[j])

    def wait_gather(j):
      pltpu.make_async_copy(feat_hbm.at[src_l.at[0]], rows[j], gsem[j]).wait()

    def start_scatter(b, j):
      pltpu.async_copy(rows[j], agg_sh.at[dst_l.at[b]], ssem[j], add=True)
      if with_count:
        pltpu.async_copy(ones_v, cnt_sh.at[dst_l.at[b]], csem, add=True)

    def wait_scatter(j):
      pltpu.make_async_copy(rows[j], agg_sh.at[dst_l.at[0]], ssem[j]).wait()

    def pipe_body(b, j, static_tail):
      jn = (j + PREF) % NBUF
      wait_gather(j)
      start_scatter(b, j)
      bn = b + PREF
      if static_tail:
        if bn < nb:
          if bn >= NBUF:
            wait_scatter(jn)
          start_gather(bn, jn)
      else:
        @pl.when(bn < nb)
        def _():
          wait_scatter(jn)
          start_gather(bn, jn)

    for b in range(PREF):
      start_gather(b, b)
    n_groups = nb // NBUF
    for j in range(NBUF):
      pipe_body(j, j, static_tail=True)

    def group(g, _):
      b0 = g * NBUF
      for j in range(NBUF):
        pipe_body(b0 + j, j, static_tail=False)
      return 0
    lax.fori_loop(1, n_groups, group, 0)
    for b in range(n_groups * NBUF, nb):
      pipe_body(b, b % NBUF, static_tail=True)

    for b in range(nb - NBUF, nb):
      wait_scatter(b % NBUF)
    if with_count:
      def cdrain(i, _):
        pltpu.make_async_copy(ones_v, cnt_sh.at[dst_l.at[0]], csem).wait()
        return 0
      lax.fori_loop(0, nb, cdrain, 0)
    plsc.subcore_barrier()

    out0 = cid * n_pad + row0
    pltpu.sync_copy(agg_sh.at[pl.ds(row0, rows_per_tile)], stage_v)
    pltpu.sync_copy(stage_v, agg_hbm.at[pl.ds(out0, rows_per_tile)])
    if with_count:
      pltpu.sync_copy(cnt_sh.at[pl.ds(row0, rows_per_tile)], cstage_v)
      pltpu.sync_copy(cstage_v, cnt_hbm.at[pl.ds(out0, rows_per_tile)])

  return pl.kernel(
      body, out_type=out_type, mesh=mesh, scratch_types=scratch,
      compiler_params=pltpu.CompilerParams(
          use_tc_tiling_on_sc=False,
          disable_bounds_checks=True,
          disable_semaphore_checks=True))


def _tc_project(np8, xp, wbig_l, wbig_r, b1_tile):
  def body(x_ref, wl_ref, wr_ref, b_ref, xl_ref, xr_ref):
    xv = x_ref[...]
    xl_ref[...] = jnp.dot(xv, wl_ref[...], preferred_element_type=jnp.float32)
    xr_ref[...] = (jnp.dot(xv, wr_ref[...], preferred_element_type=jnp.float32)
                   + b_ref[...])

  return pl.pallas_call(
      body,
      out_shape=[jax.ShapeDtypeStruct((np8, 128), jnp.float32),
                 jax.ShapeDtypeStruct((np8, 128), jnp.float32)],
  )(xp, wbig_l, wbig_r, b1_tile)


def _tc_mid(agg, cnt, xr_p, expand, w2l_bd, w2r_bd, b2_tile):
  np8 = agg.shape[1]

  def body(agg_ref, cnt_ref, xr_ref, e_ref, wl_ref, wr_ref, b_ref,
           hl_ref, hr_ref):
    a = agg_ref[0] + agg_ref[1]
    c = cnt_ref[0] + cnt_ref[1]
    inv = 1.0 / jnp.clip(c, 1.0)
    mean = a * jnp.dot(inv, e_ref[...], preferred_element_type=jnp.float32)
    hv = jnp.maximum(mean + xr_ref[...], 0.0)
    hl_ref[...] = jnp.dot(hv, wl_ref[...], preferred_element_type=jnp.float32)
    hr_ref[...] = (jnp.dot(hv, wr_ref[...], preferred_element_type=jnp.float32)
                   + b_ref[...])

  return pl.pallas_call(
      body,
      out_shape=[jax.ShapeDtypeStruct((np8, 128), jnp.float32),
                 jax.ShapeDtypeStruct((np8, 128), jnp.float32)],
  )(agg, cnt, xr_p, expand, w2l_bd, w2r_bd, b2_tile)


def _tc_post(n, c_dim, agg, cnt, hr_p, expand, gsum):
  np8 = agg.shape[1]
  hh = 128 // 8

  def body(agg_ref, cnt_ref, hr_ref, e_ref, g_ref, out_ref):
    a = agg_ref[0] + agg_ref[1]
    c = cnt_ref[0] + cnt_ref[1]
    inv = 1.0 / jnp.clip(c, 1.0)
    z = (a * jnp.dot(inv, e_ref[...], preferred_element_type=jnp.float32)
         + hr_ref[...])
    m = jnp.max(z, axis=1, keepdims=True)
    ez = jnp.exp(z - m)
    s = jnp.dot(ez, g_ref[...], preferred_element_type=jnp.float32)
    out_ref[...] = (z - m) - jnp.dot(jnp.log(s), e_ref[...],
                                     preferred_element_type=jnp.float32)

  return pl.pallas_call(
      body,
      out_shape=jax.ShapeDtypeStruct((np8, 128), jnp.float32),
  )(agg, cnt, hr_p, expand, gsum)


def kernel(x, edge_index, W1_l, W1_r, b1, W2_l, W2_r, b2):
  n, d = x.shape
  e = edge_index.shape[1]
  hh = W1_l.shape[1]
  c_dim = W2_l.shape[1]
  n_pad = ((n + NS * 16) // (NS * 16)) * (NS * 16)

  assert e % EDGE_BATCH == 0
  rows = e // EDGE_BATCH
  rows_pad = ((rows + NW - 1) // NW) * NW
  n_extra = rows_pad - rows
  src = edge_index[0].astype(jnp.int32).reshape(rows, EDGE_BATCH)
  dst = edge_index[1].astype(jnp.int32).reshape(rows, EDGE_BATCH)
  if n_extra:
    fill = jnp.arange(n_extra * EDGE_BATCH, dtype=jnp.int32)
    src_fill = (fill % n).reshape(n_extra, EDGE_BATCH)
    dst_fill = (n + fill % (n_pad - n)).reshape(n_extra, EDGE_BATCH)
    src = jnp.concatenate([src, src_fill], axis=0)
    dst = jnp.concatenate([dst, dst_fill], axis=0)
  e_pad = rows_pad * EDGE_BATCH

  np8 = n_pad // 8
  gi = jnp.arange(8)
  eye8 = jnp.eye(8, dtype=jnp.float32)
  pad_cols = ((0, 0), (0, hh - c_dim))
  wbig_l = jnp.kron(eye8, W1_l)
  wbig_r = jnp.kron(eye8, W1_r)
  w2l_bd = jnp.kron(eye8, jnp.pad(W2_l, pad_cols))
  w2r_bd = jnp.kron(eye8, jnp.pad(W2_r, pad_cols))
  b1_tile = jnp.tile(b1, 8).reshape(1, 128)
  b2_tile = jnp.tile(jnp.pad(b2, (0, hh - c_dim)), 8).reshape(1, 128)
  lane = jnp.arange(128)
  expand = (lane[None, :] // hh == gi[:, None]).astype(jnp.float32)
  gsum = ((lane[:, None] // hh == gi[None, :])
          & (lane[:, None] % hh < c_dim)).astype(jnp.float32)

  assert n % 8 == 0 and d == 128
  xp = jnp.pad(x.reshape(n // 8, 8 * d), ((0, np8 - n // 8), (0, 0)))

  agg1_fn = _make_sc_aggregate(n_pad, e_pad, hh, with_count=True)
  agg2_fn = _make_sc_aggregate(n_pad, e_pad, hh, with_count=False)

  xl_p, xr_p = _tc_project(np8, xp, wbig_l, wbig_r, b1_tile)
  agg1, cnt = agg1_fn(xl_p.reshape(n_pad, hh), src, dst)
  agg1 = agg1.reshape(NC, np8, 128)
  cnt = cnt.reshape(NC, np8, 8)
  hl_p, hr_p = _tc_mid(agg1, cnt, xr_p, expand, w2l_bd, w2r_bd, b2_tile)
  (agg2,) = agg2_fn(hl_p.reshape(n_pad, hh), src, dst)
  agg2 = agg2.reshape(NC, np8, 128)
  out_p = _tc_post(n, c_dim, agg2, cnt, hr_p, expand, gsum)
  return out_p.reshape(n_pad, hh)[:n, :c_dim]


# speedup vs baseline: 1.3549x; 1.0603x over previous
import functools

import jax
import jax.numpy as jnp
from jax import lax
from jax.experimental import pallas as pl
from jax.experimental.pallas import tpu as pltpu
from jax.experimental.pallas import tpu_sc as plsc

NC = 2
NS = 16
NW = NC * NS
EDGE_BATCH = 128


def _make_sc_aggregate(n_pad, e, h, with_count):
  assert e % (NW * EDGE_BATCH) == 0
  nb = e // (NW * EDGE_BATCH)
  assert n_pad % (NS * 16) == 0
  rows_per_tile = n_pad // NS

  mesh = plsc.VectorSubcoreMesh(
      core_axis_name="c", subcore_axis_name="s",
      num_cores=NC, num_subcores=NS)

  NBUF = 8
  PREF = 5
  assert nb >= NBUF

  out_type = [jax.ShapeDtypeStruct((NC * n_pad, h), jnp.float32)]
  scratch = [
      pltpu.VMEM_SHARED((n_pad, h), jnp.float32),
      pltpu.VMEM((nb, EDGE_BATCH), jnp.int32),
      pltpu.VMEM((nb, EDGE_BATCH), jnp.int32),
      [pltpu.VMEM((EDGE_BATCH, h), jnp.float32) for _ in range(NBUF)],
      pltpu.VMEM((rows_per_tile, h), jnp.float32),
      [pltpu.SemaphoreType.DMA for _ in range(NBUF)],
      [pltpu.SemaphoreType.DMA for _ in range(NBUF)],
  ]
  if with_count:
    out_type.append(jax.ShapeDtypeStruct((NC * n_pad,), jnp.float32))
    scratch += [
        pltpu.VMEM_SHARED((n_pad,), jnp.float32),
        pltpu.VMEM((EDGE_BATCH,), jnp.float32),
        pltpu.VMEM((rows_per_tile,), jnp.float32),
        pltpu.SemaphoreType.DMA,
    ]

  def body(feat_hbm, src_hbm, dst_hbm, agg_hbm, *rest):
    if with_count:
      (cnt_hbm, agg_sh, src_l, dst_l, rows, stage_v, gsem, ssem,
       cnt_sh, ones_v, cstage_v, csem) = rest
    else:
      agg_sh, src_l, dst_l, rows, stage_v, gsem, ssem = rest
    cid = lax.axis_index("c")
    sid = lax.axis_index("s")
    wid = sid * NC + cid
    row0 = sid * rows_per_tile

    pltpu.sync_copy(src_hbm.at[pl.ds(wid * nb, nb)], src_l)
    pltpu.sync_copy(dst_hbm.at[pl.ds(wid * nb, nb)], dst_l)

    def zrow(i, _):
      stage_v[i] = jnp.zeros((h,), jnp.float32)
      return 0
    lax.fori_loop(0, rows_per_tile, zrow, 0)
    pltpu.sync_copy(stage_v, agg_sh.at[pl.ds(row0, rows_per_tile)])
    if with_count:
      def zc(i, _):
        cstage_v[pl.ds(i * 16, 16)] = jnp.zeros((16,), jnp.float32)
        return 0
      lax.fori_loop(0, rows_per_tile // 16, zc, 0)
      pltpu.sync_copy(cstage_v, cnt_sh.at[pl.ds(row0, rows_per_tile)])
      def oc(i, _):
        ones_v[pl.ds(i * 16, 16)] = jnp.ones((16,), jnp.float32)
        return 0
      lax.fori_loop(0, EDGE_BATCH // 16, oc, 0)
    plsc.subcore_barrier()

    def start_gather(b, j):
      pltpu.async_copy(feat_hbm.at[src_l.at[b]], rows[j], gsem[j])

    def wait_gather(j):
      pltpu.make_async_copy(feat_hbm.at[src_l.at[0]], rows[j], gsem[j]).wait()

    def start_scatter(b, j):
      pltpu.async_copy(rows[j], agg_sh.at[dst_l.at[b]], ssem[j], add=True)
      if with_count:
        pltpu.async_copy(ones_v, cnt_sh.at[dst_l.at[b]], csem, add=True)

    def wait_scatter(j):
      pltpu.make_async_copy(rows[j], agg_sh.at[dst_l.at[0]], ssem[j]).wait()

    def pipe_body(b, j, static_tail):
      jn = (j + PREF) % NBUF
      wait_gather(j)
      start_scatter(b, j)
      bn = b + PREF
      if static_tail:
        if bn < nb:
          if bn >= NBUF:
            wait_scatter(jn)
          start_gather(bn, jn)
      else:
        @pl.when(bn < nb)
        def _():
          wait_scatter(jn)
          start_gather(bn, jn)

    for b in range(PREF):
      start_gather(b, b)
    n_groups = nb // NBUF
    for j in range(NBUF):
      pipe_body(j, j, static_tail=True)

    def group(g, _):
      b0 = g * NBUF
      for j in range(NBUF):
        pipe_body(b0 + j, j, static_tail=False)
      return 0
    lax.fori_loop(1, n_groups, group, 0)
    for b in range(n_groups * NBUF, nb):
      pipe_body(b, b % NBUF, static_tail=True)

    for b in range(nb - NBUF, nb):
      wait_scatter(b % NBUF)
    if with_count:
      def cdrain(i, _):
        pltpu.make_async_copy(ones_v, cnt_sh.at[dst_l.at[0]], csem).wait()
        return 0
      lax.fori_loop(0, nb, cdrain, 0)
    plsc.subcore_barrier()

    out0 = cid * n_pad + row0
    pltpu.sync_copy(agg_sh.at[pl.ds(row0, rows_per_tile)], stage_v)
    pltpu.sync_copy(stage_v, agg_hbm.at[pl.ds(out0, rows_per_tile)])
    if with_count:
      pltpu.sync_copy(cnt_sh.at[pl.ds(row0, rows_per_tile)], cstage_v)
      pltpu.sync_copy(cstage_v, cnt_hbm.at[pl.ds(out0, rows_per_tile)])

  return pl.kernel(
      body, out_type=out_type, mesh=mesh, scratch_types=scratch,
      compiler_params=pltpu.CompilerParams(
          use_tc_tiling_on_sc=False,
          disable_bounds_checks=True,
          disable_semaphore_checks=True))


def _tc_project(np8, xp, wbig_l, wbig_r, b1_tile):
  def body(x_ref, wl_ref, wr_ref, b_ref, xl_ref, xr_ref):
    xv = x_ref[...]
    xl_ref[...] = jnp.dot(xv, wl_ref[...], preferred_element_type=jnp.float32)
    xr_ref[...] = (jnp.dot(xv, wr_ref[...], preferred_element_type=jnp.float32)
                   + b_ref[...])

  return pl.pallas_call(
      body,
      out_shape=[jax.ShapeDtypeStruct((np8, 128), jnp.float32),
                 jax.ShapeDtypeStruct((np8, 128), jnp.float32)],
  )(xp, wbig_l, wbig_r, b1_tile)


def _tc_mid(agg, cnt, xr_p, expand, w2l_bd, w2r_bd, b2_tile):
  np8 = agg.shape[1]

  def body(agg_ref, cnt_ref, xr_ref, e_ref, wl_ref, wr_ref, b_ref,
           hl_ref, hr_ref):
    a = agg_ref[0] + agg_ref[1]
    c = cnt_ref[0] + cnt_ref[1]
    inv = 1.0 / jnp.clip(c, 1.0)
    mean = a * jnp.dot(inv, e_ref[...], preferred_element_type=jnp.float32)
    hv = jnp.maximum(mean + xr_ref[...], 0.0)
    hl_ref[...] = jnp.dot(hv, wl_ref[...], preferred_element_type=jnp.float32)
    hr_ref[...] = (jnp.dot(hv, wr_ref[...], preferred_element_type=jnp.float32)
                   + b_ref[...])

  return pl.pallas_call(
      body,
      out_shape=[jax.ShapeDtypeStruct((np8, 128), jnp.float32),
                 jax.ShapeDtypeStruct((np8, 128), jnp.float32)],
  )(agg, cnt, xr_p, expand, w2l_bd, w2r_bd, b2_tile)


def _tc_post(n, c_dim, agg, cnt, hr_p, expand, gsum):
  np8 = agg.shape[1]
  hh = 128 // 8

  def body(agg_ref, cnt_ref, hr_ref, e_ref, g_ref, out_ref):
    a = agg_ref[0] + agg_ref[1]
    c = cnt_ref[0] + cnt_ref[1]
    inv = 1.0 / jnp.clip(c, 1.0)
    z = (a * jnp.dot(inv, e_ref[...], preferred_element_type=jnp.float32)
         + hr_ref[...])
    m = jnp.max(z, axis=1, keepdims=True)
    ez = jnp.exp(z - m)
    s = jnp.dot(ez, g_ref[...], preferred_element_type=jnp.float32)
    out_ref[...] = (z - m) - jnp.dot(jnp.log(s), e_ref[...],
                                     preferred_element_type=jnp.float32)

  return pl.pallas_call(
      body,
      out_shape=jax.ShapeDtypeStruct((np8, 128), jnp.float32),
  )(agg, cnt, hr_p, expand, gsum)


def kernel(x, edge_index, W1_l, W1_r, b1, W2_l, W2_r, b2):
  n, d = x.shape
  e = edge_index.shape[1]
  hh = W1_l.shape[1]
  c_dim = W2_l.shape[1]
  n_pad = ((n + NS * 16) // (NS * 16)) * (NS * 16)

  assert e % EDGE_BATCH == 0
  rows = e // EDGE_BATCH
  rows_pad = ((rows + NW - 1) // NW) * NW
  n_extra = rows_pad - rows
  er = edge_index.astype(jnp.int32).reshape(2, rows, EDGE_BATCH)
  src = er[0]
  dst = er[1]
  if n_extra:
    fill = jnp.arange(n_extra * EDGE_BATCH, dtype=jnp.int32)
    src_fill = (fill % n).reshape(n_extra, EDGE_BATCH)
    dst_fill = (n + fill % (n_pad - n)).reshape(n_extra, EDGE_BATCH)
    src = jnp.concatenate([src, src_fill], axis=0)
    dst = jnp.concatenate([dst, dst_fill], axis=0)
  e_pad = rows_pad * EDGE_BATCH

  np8 = n_pad // 8
  gi = jnp.arange(8)
  eye8 = jnp.eye(8, dtype=jnp.float32)
  pad_cols = ((0, 0), (0, hh - c_dim))
  wbig_l = jnp.kron(eye8, W1_l)
  wbig_r = jnp.kron(eye8, W1_r)
  w2l_bd = jnp.kron(eye8, jnp.pad(W2_l, pad_cols))
  w2r_bd = jnp.kron(eye8, jnp.pad(W2_r, pad_cols))
  b1_tile = jnp.tile(b1, 8).reshape(1, 128)
  b2_tile = jnp.tile(jnp.pad(b2, (0, hh - c_dim)), 8).reshape(1, 128)
  lane = jnp.arange(128)
  expand = (lane[None, :] // hh == gi[:, None]).astype(jnp.float32)
  gsum = ((lane[:, None] // hh == gi[None, :])
          & (lane[:, None] % hh < c_dim)).astype(jnp.float32)

  assert n % 8 == 0 and d == 128
  xp = jnp.pad(x.reshape(n // 8, 8 * d), ((0, np8 - n // 8), (0, 0)))

  agg1_fn = _make_sc_aggregate(n_pad, e_pad, hh, with_count=True)
  agg2_fn = _make_sc_aggregate(n_pad, e_pad, hh, with_count=False)

  xl_p, xr_p = _tc_project(np8, xp, wbig_l, wbig_r, b1_tile)
  agg1, cnt = agg1_fn(xl_p.reshape(n_pad, hh), src, dst)
  agg1 = agg1.reshape(NC, np8, 128)
  cnt = cnt.reshape(NC, np8, 8)
  hl_p, hr_p = _tc_mid(agg1, cnt, xr_p, expand, w2l_bd, w2r_bd, b2_tile)
  (agg2,) = agg2_fn(hl_p.reshape(n_pad, hh), src, dst)
  agg2 = agg2.reshape(NC, np8, 128)
  out_p = _tc_post(n, c_dim, agg2, cnt, hr_p, expand, gsum)
  return out_p.reshape(n_pad, hh)[:n, :c_dim]


# speedup vs baseline: 1.3552x; 1.0002x over previous
import jax
import jax.numpy as jnp
from jax import lax
from jax.experimental import pallas as pl
from jax.experimental.pallas import tpu as pltpu
from jax.experimental.pallas import tpu_sc as plsc

NC = 2
NS = 16
NW = NC * NS
EDGE_BATCH = 128


def _make_sc_aggregate(n_pad, e, h, with_count):
  assert e % (NW * EDGE_BATCH) == 0
  nb = e // (NW * EDGE_BATCH)
  assert n_pad % (NS * 16) == 0
  rows_per_tile = n_pad // NS

  mesh = plsc.VectorSubcoreMesh(
      core_axis_name="c", subcore_axis_name="s",
      num_cores=NC, num_subcores=NS)

  NBUF = 8
  PREF = 5
  assert nb >= NBUF

  out_type = [jax.ShapeDtypeStruct((NC * n_pad, h), jnp.float32)]
  scratch = [
      pltpu.VMEM_SHARED((n_pad, h), jnp.float32),
      pltpu.VMEM((nb, EDGE_BATCH), jnp.int32),
      pltpu.VMEM((nb, EDGE_BATCH), jnp.int32),
      [pltpu.VMEM((EDGE_BATCH, h), jnp.float32) for _ in range(NBUF)],
      pltpu.VMEM((rows_per_tile, h), jnp.float32),
      [pltpu.SemaphoreType.DMA for _ in range(NBUF)],
      [pltpu.SemaphoreType.DMA for _ in range(NBUF)],
  ]
  if with_count:
    out_type.append(jax.ShapeDtypeStruct((NC * n_pad,), jnp.float32))
    scratch += [
        pltpu.VMEM_SHARED((n_pad,), jnp.float32),
        pltpu.VMEM((EDGE_BATCH,), jnp.float32),
        pltpu.VMEM((rows_per_tile,), jnp.float32),
        pltpu.SemaphoreType.DMA,
    ]

  def body(feat_hbm, src_hbm, dst_hbm, agg_hbm, *rest):
    if with_count:
      (cnt_hbm, agg_sh, src_l, dst_l, rows, stage_v, gsem, ssem,
       cnt_sh, ones_v, cstage_v, csem) = rest
    else:
      agg_sh, src_l, dst_l, rows, stage_v, gsem, ssem = rest
    cid = lax.axis_index("c")
    sid = lax.axis_index("s")
    wid = sid * NC + cid
    row0 = sid * rows_per_tile

    pltpu.sync_copy(src_hbm.at[pl.ds(wid * nb, nb)], src_l)
    pltpu.sync_copy(dst_hbm.at[pl.ds(wid * nb, nb)], dst_l)

    def zrow(i, _):
      stage_v[i] = jnp.zeros((h,), jnp.float32)
      return 0
    lax.fori_loop(0, rows_per_tile, zrow, 0)
    pltpu.sync_copy(stage_v, agg_sh.at[pl.ds(row0, rows_per_tile)])
    if with_count:
      def zc(i, _):
        cstage_v[pl.ds(i * 16, 16)] = jnp.zeros((16,), jnp.float32)
        return 0
      lax.fori_loop(0, rows_per_tile // 16, zc, 0)
      pltpu.sync_copy(cstage_v, cnt_sh.at[pl.ds(row0, rows_per_tile)])
      def oc(i, _):
        ones_v[pl.ds(i * 16, 16)] = jnp.ones((16,), jnp.float32)
        return 0
      lax.fori_loop(0, EDGE_BATCH // 16, oc, 0)
    plsc.subcore_barrier()

    def start_gather(b, j):
      pltpu.async_copy(feat_hbm.at[src_l.at[b]], rows[j], gsem[j])

    def wait_gather(j):
      pltpu.make_async_copy(feat_hbm.at[src_l.at[0]], rows[j], gsem[j]).wait()

    def start_scatter(b, j):
      pltpu.async_copy(rows[j], agg_sh.at[dst_l.at[b]], ssem[j], add=True)
      if with_count:
        pltpu.async_copy(ones_v, cnt_sh.at[dst_l.at[b]], csem, add=True)

    def wait_scatter(j):
      pltpu.make_async_copy(rows[j], agg_sh.at[dst_l.at[0]], ssem[j]).wait()

    def pipe_body(b, j, static_tail):
      jn = (j + PREF) % NBUF
      wait_gather(j)
      start_scatter(b, j)
      bn = b + PREF
      if static_tail:
        if bn < nb:
          if bn >= NBUF:
            wait_scatter(jn)
          start_gather(bn, jn)
      else:
        @pl.when(bn < nb)
        def _():
          wait_scatter(jn)
          start_gather(bn, jn)

    for b in range(PREF):
      start_gather(b, b)
    n_groups = nb // NBUF
    for j in range(NBUF):
      pipe_body(j, j, static_tail=True)

    def group(g, _):
      b0 = g * NBUF
      for j in range(NBUF):
        pipe_body(b0 + j, j, static_tail=False)
      return 0
    lax.fori_loop(1, n_groups, group, 0)
    for b in range(n_groups * NBUF, nb):
      pipe_body(b, b % NBUF, static_tail=True)

    for b in range(nb - NBUF, nb):
      wait_scatter(b % NBUF)
    if with_count:
      def cdrain(i, _):
        pltpu.make_async_copy(ones_v, cnt_sh.at[dst_l.at[0]], csem).wait()
        return 0
      lax.fori_loop(0, nb, cdrain, 0)
    plsc.subcore_barrier()

    out0 = cid * n_pad + row0
    pltpu.sync_copy(agg_sh.at[pl.ds(row0, rows_per_tile)], stage_v)
    pltpu.sync_copy(stage_v, agg_hbm.at[pl.ds(out0, rows_per_tile)])
    if with_count:
      pltpu.sync_copy(cnt_sh.at[pl.ds(row0, rows_per_tile)], cstage_v)
      pltpu.sync_copy(cstage_v, cnt_hbm.at[pl.ds(out0, rows_per_tile)])

  return pl.kernel(
      body, out_type=out_type, mesh=mesh, scratch_types=scratch,
      compiler_params=pltpu.CompilerParams(
          use_tc_tiling_on_sc=False,
          disable_bounds_checks=True,
          disable_semaphore_checks=True))


def _tc_project(np8, xp, wbig_l, wbig_r, b1_tile):
  def body(x_ref, wl_ref, wr_ref, b_ref, xl_ref, xr_ref):
    xv = x_ref[...]
    xl_ref[...] = jnp.dot(xv, wl_ref[...], preferred_element_type=jnp.float32)
    xr_ref[...] = (jnp.dot(xv, wr_ref[...], preferred_element_type=jnp.float32)
                   + b_ref[...])

  return pl.pallas_call(
      body,
      out_shape=[jax.ShapeDtypeStruct((np8, 128), jnp.float32),
                 jax.ShapeDtypeStruct((np8, 128), jnp.float32)],
  )(xp, wbig_l, wbig_r, b1_tile)


def _tc_mid(agg, cnt, xr_p, expand, w2l_bd, w2r_bd, b2_tile):
  np8 = agg.shape[1]

  def body(agg_ref, cnt_ref, xr_ref, e_ref, wl_ref, wr_ref, b_ref,
           hl_ref, hr_ref):
    a = agg_ref[0] + agg_ref[1]
    c = cnt_ref[0] + cnt_ref[1]
    inv = 1.0 / jnp.clip(c, 1.0)
    mean = a * jnp.dot(inv, e_ref[...], preferred_element_type=jnp.float32)
    hv = jnp.maximum(mean + xr_ref[...], 0.0)
    hl_ref[...] = jnp.dot(hv, wl_ref[...], preferred_element_type=jnp.float32)
    hr_ref[...] = (jnp.dot(hv, wr_ref[...], preferred_element_type=jnp.float32)
                   + b_ref[...])

  return pl.pallas_call(
      body,
      out_shape=[jax.ShapeDtypeStruct((np8, 128), jnp.float32),
                 jax.ShapeDtypeStruct((np8, 128), jnp.float32)],
  )(agg, cnt, xr_p, expand, w2l_bd, w2r_bd, b2_tile)


def _tc_post(n, c_dim, agg, cnt, hr_p, expand, gsum):
  np8 = agg.shape[1]

  def body(agg_ref, cnt_ref, hr_ref, e_ref, g_ref, out_ref):
    a = agg_ref[0] + agg_ref[1]
    c = cnt_ref[0] + cnt_ref[1]
    inv = 1.0 / jnp.clip(c, 1.0)
    z = (a * jnp.dot(inv, e_ref[...], preferred_element_type=jnp.float32)
         + hr_ref[...])
    m = jnp.max(z, axis=1, keepdims=True)
    ez = jnp.exp(z - m)
    s = jnp.dot(ez, g_ref[...], preferred_element_type=jnp.float32)
    out_ref[...] = (z - m) - jnp.dot(jnp.log(s), e_ref[...],
                                     preferred_element_type=jnp.float32)

  return pl.pallas_call(
      body,
      out_shape=jax.ShapeDtypeStruct((np8, 128), jnp.float32),
  )(agg, cnt, hr_p, expand, gsum)


def kernel(x, edge_index, W1_l, W1_r, b1, W2_l, W2_r, b2):
  n, d = x.shape
  e = edge_index.shape[1]
  hh = W1_l.shape[1]
  c_dim = W2_l.shape[1]
  n_pad = ((n + NS * 16) // (NS * 16)) * (NS * 16)

  assert e % EDGE_BATCH == 0
  rows = e // EDGE_BATCH
  rows_pad = ((rows + NW - 1) // NW) * NW
  n_extra = rows_pad - rows
  er = edge_index.astype(jnp.int32).reshape(2, rows, EDGE_BATCH)
  src = er[0]
  dst = er[1]
  if n_extra:
    fill = jnp.arange(n_extra * EDGE_BATCH, dtype=jnp.int32)
    src_fill = (fill % n).reshape(n_extra, EDGE_BATCH)
    dst_fill = (n + fill % (n_pad - n)).reshape(n_extra, EDGE_BATCH)
    src = jnp.concatenate([src, src_fill], axis=0)
    dst = jnp.concatenate([dst, dst_fill], axis=0)
  e_pad = rows_pad * EDGE_BATCH

  np8 = n_pad // 8
  gi = jnp.arange(8)
  eye8 = jnp.eye(8, dtype=jnp.float32)
  pad_cols = ((0, 0), (0, hh - c_dim))
  wbig_l = jnp.kron(eye8, W1_l)
  wbig_r = jnp.kron(eye8, W1_r)
  w2l_bd = jnp.kron(eye8, jnp.pad(W2_l, pad_cols))
  w2r_bd = jnp.kron(eye8, jnp.pad(W2_r, pad_cols))
  b1_tile = jnp.tile(b1, 8).reshape(1, 128)
  b2_tile = jnp.tile(jnp.pad(b2, (0, hh - c_dim)), 8).reshape(1, 128)
  lane = jnp.arange(128)
  expand = (lane[None, :] // hh == gi[:, None]).astype(jnp.float32)
  gsum = ((lane[:, None] // hh == gi[None, :])
          & (lane[:, None] % hh < c_dim)).astype(jnp.float32)

  assert n % 8 == 0 and d == 128
  xp = jnp.pad(x.reshape(n // 8, 8 * d), ((0, np8 - n // 8), (0, 0)))

  agg1_fn = _make_sc_aggregate(n_pad, e_pad, hh, with_count=True)
  agg2_fn = _make_sc_aggregate(n_pad, e_pad, hh, with_count=False)

  xl_p, xr_p = _tc_project(np8, xp, wbig_l, wbig_r, b1_tile)
  agg1, cnt = agg1_fn(xl_p.reshape(n_pad, hh), src, dst)
  agg1 = agg1.reshape(NC, np8, 128)
  cnt = cnt.reshape(NC, np8, 8)
  hl_p, hr_p = _tc_mid(agg1, cnt, xr_p, expand, w2l_bd, w2r_bd, b2_tile)
  (agg2,) = agg2_fn(hl_p.reshape(n_pad, hh), src, dst)
  agg2 = agg2.reshape(NC, np8, 128)
  out_p = _tc_post(n, c_dim, agg2, cnt, hr_p, expand, gsum)
  return out_p.reshape(n_pad, hh)[:n, :c_dim]
